# baseline (speedup 1.0000x reference)
import functools

import jax
import jax.numpy as jnp
from jax import lax
from jax.experimental import pallas as pl
from jax.experimental.pallas import tpu as pltpu
from jax.experimental.pallas import tpu_sc as plsc

VOCAB = 100000
HID = 768
B = 4
S = 2048
EPS = 1e-12

NC = 2
NS = 16
NW = NC * NS
L = 16
NSL = HID // L

POS_PER_W = S // NW
PC = 16
NCHUNK = POS_PER_W // PC
ROWS_PER_CHUNK = B * PC


def _rsqrt_vec(v):
    i = lax.bitcast_convert_type(v, jnp.int32)
    i = jnp.int32(0x5F3759DF) - lax.shift_right_arithmetic(i, jnp.int32(1))
    y = lax.bitcast_convert_type(i, jnp.float32)
    half = v * jnp.float32(-0.5)
    for _ in range(3):
        y = y * (jnp.float32(1.5) + half * y * y)
    return y


def _body(ids_hbm, tt_hbm, word_hbm, pos_hbm, type_hbm, gamma_hbm, beta_hbm,
          out_hbm, idx_v, ttv_v, rows_v, pos_v, type_v, gamma_v, beta_v, sem):
    wid = lax.axis_index("s") * NC + lax.axis_index("c")

    pltpu.sync_copy(type_hbm, type_v)
    pltpu.sync_copy(gamma_hbm, gamma_v)
    pltpu.sync_copy(beta_hbm, beta_v)

    def chunk_body(c, _):
        pbase = wid * POS_PER_W + c * PC
        for b in range(B):
            pltpu.sync_copy(ids_hbm.at[pl.ds(b * S + pbase, PC)],
                            idx_v.at[pl.ds(b * PC, PC)])
            pltpu.sync_copy(tt_hbm.at[pl.ds(b * S + pbase, PC)],
                            ttv_v.at[pl.ds(b * PC, PC)])
        gather = pltpu.async_copy(word_hbm.at[idx_v], rows_v, sem)
        pltpu.sync_copy(pos_hbm.at[pl.ds(pbase, PC)], pos_v)
        gather.wait()

        def row_body(r, _):
            tt = ttv_v[pl.ds(r, L)][0]
            pi = lax.rem(r, PC)
            acc = jnp.zeros((L,), jnp.float32)
            acc2 = jnp.zeros((L,), jnp.float32)
            for h in range(NSL):
                ds = pl.ds(h * L, L)
                x = rows_v[r, ds] + pos_v[pi, ds] + type_v[tt, ds]
                rows_v[r, ds] = x
                acc = acc + x
                acc2 = acc2 + x * x
            mean = jnp.sum(acc) * jnp.float32(1.0 / HID)
            var = jnp.sum(acc2) * jnp.float32(1.0 / HID) - mean * mean
            inv = _rsqrt_vec(jnp.full((L,), var + jnp.float32(EPS)))
            mean_v = jnp.full((L,), mean)
            for h in range(NSL):
                ds = pl.ds(h * L, L)
                g2 = gamma_v[ds] * inv
                rows_v[r, ds] = rows_v[r, ds] * g2 + (beta_v[ds] - mean_v * g2)
            return 0

        lax.fori_loop(0, ROWS_PER_CHUNK, row_body, 0)

        for b in range(B):
            pltpu.sync_copy(rows_v.at[pl.ds(b * PC, PC)],
                            out_hbm.at[pl.ds(b * S + pbase, PC)])
        return 0

    lax.fori_loop(0, NCHUNK, chunk_body, 0)


@jax.jit
def _run(input_ids_flat, token_type_flat, word_emb, pos_emb, type_emb,
         gamma, beta):
    mesh = plsc.VectorSubcoreMesh(core_axis_name="c", subcore_axis_name="s")
    out = pl.kernel(
        _body,
        out_type=jax.ShapeDtypeStruct((B * S, HID), jnp.float32),
        mesh=mesh,
        compiler_params=pltpu.CompilerParams(needs_layout_passes=False),
        scratch_types=[
            pltpu.VMEM((ROWS_PER_CHUNK,), jnp.int32),
            pltpu.VMEM((ROWS_PER_CHUNK + L,), jnp.int32),
            pltpu.VMEM((ROWS_PER_CHUNK, HID), jnp.float32),
            pltpu.VMEM((PC, HID), jnp.float32),
            pltpu.VMEM((2, HID), jnp.float32),
            pltpu.VMEM((HID,), jnp.float32),
            pltpu.VMEM((HID,), jnp.float32),
            pltpu.SemaphoreType.DMA,
        ],
    )(input_ids_flat, token_type_flat, word_emb, pos_emb, type_emb,
      gamma, beta)
    return out


def kernel(input_ids, token_type_ids, word_emb, pos_emb, type_emb, gamma,
           beta):
    ids_flat = input_ids.reshape(-1).astype(jnp.int32)
    tt_flat = token_type_ids.reshape(-1).astype(jnp.int32)
    out = _run(ids_flat, tt_flat, word_emb, pos_emb, type_emb, gamma, beta)
    return out.reshape(B, S, HID)


# speedup vs baseline: 1.8669x; 1.8669x over previous
import jax
import jax.numpy as jnp
from jax import lax
from jax.experimental import pallas as pl
from jax.experimental.pallas import tpu as pltpu
from jax.experimental.pallas import tpu_sc as plsc

VOCAB = 100000
HID = 768
B = 4
S = 2048
EPS = 1e-12

NC = 2
NS = 16
NW = NC * NS
L = 16
NSL = HID // L

POS_PER_W = S // NW
PC = 16
NCHUNK = POS_PER_W // PC
RPC = B * PC
UNROLL = 8


def _rsqrt_vec(v):
    i = lax.bitcast_convert_type(v, jnp.int32)
    i = jnp.int32(0x5F3759DF) - lax.shift_right_arithmetic(i, jnp.int32(1))
    y = lax.bitcast_convert_type(i, jnp.float32)
    half = v * jnp.float32(-0.5)
    for _ in range(3):
        y = y * (jnp.float32(1.5) + half * y * y)
    return y


def _body(ids_hbm, tt_hbm, word_hbm, pos_hbm, type_hbm, gamma_hbm, beta_hbm,
          out_hbm, idx_all, tt_all, rows0, rows1, pos0, pos1, type_v, diff_v,
          gamma_v, beta_v, gsem0, gsem1, psem0, psem1, ssem):
    rows = (rows0, rows1)
    posb = (pos0, pos1)
    gsem = (gsem0, gsem1)
    psem = (psem0, psem1)

    wid = lax.axis_index("s") * NC + lax.axis_index("c")
    base = wid * POS_PER_W

    pltpu.sync_copy(type_hbm, type_v)
    pltpu.sync_copy(gamma_hbm, gamma_v)
    pltpu.sync_copy(beta_hbm, beta_v)

    for c in range(NCHUNK):
        for b in range(B):
            src = pl.ds(b * S + base + c * PC, PC)
            dst = pl.ds(c * RPC + b * PC, PC)
            pltpu.async_copy(ids_hbm.at[src], idx_all.at[dst], ssem)
            pltpu.async_copy(tt_hbm.at[src], tt_all.at[dst], ssem)
    pltpu.make_async_copy(ids_hbm.at[pl.ds(0, NCHUNK * RPC)], idx_all,
                          ssem).wait()
    pltpu.make_async_copy(
        tt_hbm.at[pl.ds(0, NCHUNK * RPC)],
        tt_all.at[pl.ds(0, NCHUNK * RPC)], ssem).wait()

    for h in range(NSL):
        ds = pl.ds(h * L, L)
        diff_v[ds] = type_v[1, ds] - type_v[0, ds]

    def start_chunk(c):
        k = c % 2
        g = pltpu.async_copy(word_hbm.at[idx_all.at[pl.ds(c * RPC, RPC)]],
                             rows[k], gsem[k])
        p = pltpu.async_copy(pos_hbm.at[pl.ds(base + c * PC, PC)], posb[k],
                             psem[k])
        return g, p

    def compute_chunk(c):
        k = c % 2
        rows_k = rows[k]
        pos_k = posb[k]

        def group(pi, _):
            ttv = []
            for b in range(B):
                tt = tt_all[pl.ds(c * RPC + b * PC + pi, L)][0]
                ttv.append(jnp.full((L,), tt.astype(jnp.float32)))

            def sl(h, carry):
                acc = list(carry[:B])
                acc2 = list(carry[B:])
                ds = pl.ds(h * L, L)
                pt = pos_k[pi, ds] + type_v[0, ds]
                d = diff_v[ds]
                for b in range(B):
                    x = rows_k[b * PC + pi, ds] + pt + ttv[b] * d
                    rows_k[b * PC + pi, ds] = x
                    acc[b] = acc[b] + x
                    acc2[b] = acc2[b] + x * x
                return tuple(acc) + tuple(acc2)

            z = jnp.zeros((L,), jnp.float32)
            carry = lax.fori_loop(0, NSL, sl, (z,) * (2 * B), unroll=UNROLL)

            inv = []
            miv = []
            for b in range(B):
                mean = jnp.sum(carry[b]) * jnp.float32(1.0 / HID)
                var = (jnp.sum(carry[B + b]) * jnp.float32(1.0 / HID)
                       - mean * mean)
                iv = _rsqrt_vec(jnp.full((L,), var + jnp.float32(EPS)))
                inv.append(iv)
                miv.append(jnp.full((L,), mean) * iv)

            def sl2(h, _):
                ds = pl.ds(h * L, L)
                g = gamma_v[ds]
                bt = beta_v[ds]
                for b in range(B):
                    x = rows_k[b * PC + pi, ds]
                    rows_k[b * PC + pi, ds] = (x * inv[b] - miv[b]) * g + bt
                return 0

            lax.fori_loop(0, NSL, sl2, 0, unroll=UNROLL)
            return 0

        lax.fori_loop(0, PC, group, 0)

        for b in range(B):
            pltpu.sync_copy(rows_k.at[pl.ds(b * PC, PC)],
                            out_hbm.at[pl.ds(b * S + base + c * PC, PC)])

    pend = {}
    for c in range(2):
        pend[c] = start_chunk(c)
    for c in range(NCHUNK):
        g, p = pend.pop(c)
        g.wait()
        p.wait()
        compute_chunk(c)
        if c + 2 < NCHUNK:
            pend[c + 2] = start_chunk(c + 2)


@jax.jit
def _run(input_ids_flat, token_type_flat, word_emb, pos_emb, type_emb,
         gamma, beta):
    mesh = plsc.VectorSubcoreMesh(core_axis_name="c", subcore_axis_name="s")
    out = pl.kernel(
        _body,
        out_type=jax.ShapeDtypeStruct((B * S, HID), jnp.float32),
        mesh=mesh,
        compiler_params=pltpu.CompilerParams(needs_layout_passes=False),
        scratch_types=[
            pltpu.VMEM((NCHUNK * RPC,), jnp.int32),
            pltpu.VMEM((NCHUNK * RPC + L,), jnp.int32),
            pltpu.VMEM((RPC, HID), jnp.float32),
            pltpu.VMEM((RPC, HID), jnp.float32),
            pltpu.VMEM((PC, HID), jnp.float32),
            pltpu.VMEM((PC, HID), jnp.float32),
            pltpu.VMEM((2, HID), jnp.float32),
            pltpu.VMEM((HID,), jnp.float32),
            pltpu.VMEM((HID,), jnp.float32),
            pltpu.VMEM((HID,), jnp.float32),
            pltpu.SemaphoreType.DMA,
            pltpu.SemaphoreType.DMA,
            pltpu.SemaphoreType.DMA,
            pltpu.SemaphoreType.DMA,
            pltpu.SemaphoreType.DMA,
        ],
    )(input_ids_flat, token_type_flat, word_emb, pos_emb, type_emb,
      gamma, beta)
    return out


def kernel(input_ids, token_type_ids, word_emb, pos_emb, type_emb, gamma,
           beta):
    ids_flat = input_ids.reshape(-1).astype(jnp.int32)
    tt_flat = token_type_ids.reshape(-1).astype(jnp.int32)
    out = _run(ids_flat, tt_flat, word_emb, pos_emb, type_emb, gamma, beta)
    return out.reshape(B, S, HID)
